# baseline (speedup 1.0000x reference)
import functools

import jax
import jax.numpy as jnp
from jax import lax
from jax.experimental import pallas as pl
from jax.experimental.pallas import tpu as pltpu
from jax.experimental.pallas import tpu_sc as plsc

_WCHUNK = 1 << 19


def _sc_body(data_ref, x_ref, out_ref, hbuf, sem, hsem):
    c = lax.axis_index("c")
    s = lax.axis_index("s")
    wid = s * 2 + c

    @pl.when(wid == 0)
    def _():
        pltpu.make_async_copy(x_ref, hbuf, hsem).start()
        bulk = pltpu.make_async_copy(
            data_ref.at[pl.ds(0, _WCHUNK - 128)],
            out_ref.at[pl.ds(128, _WCHUNK - 128)],
            sem,
        )
        bulk.start()
        pltpu.make_async_copy(x_ref, hbuf, hsem).wait()
        hstore = pltpu.make_async_copy(hbuf, out_ref.at[pl.ds(0, 128)], hsem)
        hstore.start()
        bulk.wait()
        hstore.wait()

    @pl.when(wid != 0)
    def _():
        base = pl.multiple_of(wid * _WCHUNK, 1 << 19)
        bulk = pltpu.make_async_copy(
            data_ref.at[pl.ds(base - 128, _WCHUNK)],
            out_ref.at[pl.ds(base, _WCHUNK)],
            sem,
        )
        bulk.start()
        bulk.wait()


def kernel(data, x):
    n, d = data.shape
    mesh = plsc.VectorSubcoreMesh(core_axis_name="c", subcore_axis_name="s")
    sc_fn = functools.partial(
        pl.kernel,
        mesh=mesh,
        out_type=jax.ShapeDtypeStruct((n * d,), data.dtype),
        scratch_types=[
            pltpu.VMEM((128,), jnp.float32),
            pltpu.SemaphoreType.DMA,
            pltpu.SemaphoreType.DMA,
        ],
    )(_sc_body)
    flat = sc_fn(data.reshape(-1), x)
    return flat.reshape(n, d)


# speedup vs baseline: 30.1081x; 30.1081x over previous
import jax
import jax.numpy as jnp
from jax.experimental import pallas as pl
from jax.experimental.pallas import tpu as pltpu

_C = 1 << 19
_NBUF = 8


def _shift_body(data_ref, x_ref, out_ref, bufs, lsems, ssems, hsem):
    total = data_ref.shape[0] - 128
    nc = (total + _C - 1) // _C

    def load(k):
        off = k * _C
        sz = min(_C, total - off)
        b = k % _NBUF
        return pltpu.make_async_copy(
            data_ref.at[pl.ds(off, sz)],
            bufs.at[b, pl.ds(0, sz)],
            lsems.at[b],
        )

    def store(k):
        off = k * _C
        sz = min(_C, total - off)
        b = k % _NBUF
        return pltpu.make_async_copy(
            bufs.at[b, pl.ds(0, sz)],
            out_ref.at[pl.ds(128 + off, sz)],
            ssems.at[b],
        )

    loads = [load(k) for k in range(nc)]
    stores = [store(k) for k in range(nc)]

    head = pltpu.make_async_copy(x_ref, out_ref.at[pl.ds(0, 128)], hsem)
    head.start()

    for k in range(min(_NBUF, nc)):
        loads[k].start(priority=k % 2)
    for k in range(nc):
        loads[k].wait()
        stores[k].start(priority=k % 2)
        nl = k + 1
        if _NBUF <= nl < nc:
            stores[nl - _NBUF].wait()
            loads[nl].start(priority=nl % 2)
    for k in range(max(0, nc - _NBUF), nc):
        stores[k].wait()
    head.wait()


def kernel(data, x):
    n, d = data.shape
    flat = pl.pallas_call(
        _shift_body,
        in_specs=[
            pl.BlockSpec(memory_space=pl.ANY),
            pl.BlockSpec(memory_space=pl.ANY),
        ],
        out_specs=pl.BlockSpec(memory_space=pl.ANY),
        out_shape=jax.ShapeDtypeStruct((n * d,), data.dtype),
        scratch_shapes=[
            pltpu.VMEM((_NBUF, _C), jnp.float32),
            pltpu.SemaphoreType.DMA((_NBUF,)),
            pltpu.SemaphoreType.DMA((_NBUF,)),
            pltpu.SemaphoreType.DMA,
        ],
    )(data.reshape(-1), x)
    return flat.reshape(n, d)


# speedup vs baseline: 36.4870x; 1.2119x over previous
import jax
import jax.numpy as jnp
from jax.experimental import pallas as pl
from jax.experimental.pallas import tpu as pltpu

_C = 1 << 20
_NBUF = 4


def _shift_body(data_ref, x_ref, out_ref, bufs, lsems, ssems, hsem):
    total = data_ref.shape[0] - 128
    nc = (total + _C - 1) // _C

    def load(k):
        off = k * _C
        sz = min(_C, total - off)
        b = k % _NBUF
        return pltpu.make_async_copy(
            data_ref.at[pl.ds(off, sz)],
            bufs.at[b, pl.ds(0, sz)],
            lsems.at[b],
        )

    def store(k):
        off = k * _C
        sz = min(_C, total - off)
        b = k % _NBUF
        return pltpu.make_async_copy(
            bufs.at[b, pl.ds(0, sz)],
            out_ref.at[pl.ds(128 + off, sz)],
            ssems.at[b],
        )

    loads = [load(k) for k in range(nc)]
    stores = [store(k) for k in range(nc)]

    head = pltpu.make_async_copy(x_ref, out_ref.at[pl.ds(0, 128)], hsem)
    head.start()

    for k in range(min(_NBUF, nc)):
        loads[k].start(priority=k % 2)
    for k in range(nc):
        loads[k].wait()
        stores[k].start(priority=k % 2)
        nl = k + 1
        if _NBUF <= nl < nc:
            stores[nl - _NBUF].wait()
            loads[nl].start(priority=nl % 2)
    for k in range(max(0, nc - _NBUF), nc):
        stores[k].wait()
    head.wait()


def kernel(data, x):
    n, d = data.shape
    flat = pl.pallas_call(
        _shift_body,
        in_specs=[
            pl.BlockSpec(memory_space=pl.ANY),
            pl.BlockSpec(memory_space=pl.ANY),
        ],
        out_specs=pl.BlockSpec(memory_space=pl.ANY),
        out_shape=jax.ShapeDtypeStruct((n * d,), data.dtype),
        scratch_shapes=[
            pltpu.VMEM((_NBUF, _C), jnp.float32),
            pltpu.SemaphoreType.DMA((_NBUF,)),
            pltpu.SemaphoreType.DMA((_NBUF,)),
            pltpu.SemaphoreType.DMA,
        ],
    )(data.reshape(-1), x)
    return flat.reshape(n, d)


# speedup vs baseline: 45.2522x; 1.2402x over previous
import jax
import jax.numpy as jnp
from jax.experimental import pallas as pl
from jax.experimental.pallas import tpu as pltpu

_C = 1 << 21
_NBUF = 4


def _shift_body(data_ref, x_ref, out_ref, bufs, lsems, ssems, hsem):
    total = data_ref.shape[0] - 128
    nc = (total + _C - 1) // _C

    def load(k):
        off = k * _C
        sz = min(_C, total - off)
        b = k % _NBUF
        return pltpu.make_async_copy(
            data_ref.at[pl.ds(off, sz)],
            bufs.at[b, pl.ds(0, sz)],
            lsems.at[b],
        )

    def store(k):
        off = k * _C
        sz = min(_C, total - off)
        b = k % _NBUF
        return pltpu.make_async_copy(
            bufs.at[b, pl.ds(0, sz)],
            out_ref.at[pl.ds(128 + off, sz)],
            ssems.at[b],
        )

    loads = [load(k) for k in range(nc)]
    stores = [store(k) for k in range(nc)]

    head = pltpu.make_async_copy(x_ref, out_ref.at[pl.ds(0, 128)], hsem)
    head.start()

    for k in range(min(_NBUF, nc)):
        loads[k].start(priority=k % 2)
    for k in range(nc):
        loads[k].wait()
        stores[k].start(priority=k % 2)
        nl = k + 1
        if _NBUF <= nl < nc:
            stores[nl - _NBUF].wait()
            loads[nl].start(priority=nl % 2)
    for k in range(max(0, nc - _NBUF), nc):
        stores[k].wait()
    head.wait()


def kernel(data, x):
    n, d = data.shape
    flat = pl.pallas_call(
        _shift_body,
        in_specs=[
            pl.BlockSpec(memory_space=pl.ANY),
            pl.BlockSpec(memory_space=pl.ANY),
        ],
        out_specs=pl.BlockSpec(memory_space=pl.ANY),
        out_shape=jax.ShapeDtypeStruct((n * d,), data.dtype),
        scratch_shapes=[
            pltpu.VMEM((_NBUF, _C), jnp.float32),
            pltpu.SemaphoreType.DMA((_NBUF,)),
            pltpu.SemaphoreType.DMA((_NBUF,)),
            pltpu.SemaphoreType.DMA,
        ],
    )(data.reshape(-1), x)
    return flat.reshape(n, d)
